# baseline (speedup 1.0000x reference)
import functools

import jax
import jax.numpy as jnp
from jax import lax
from jax.experimental import pallas as pl
from jax.experimental.pallas import tpu as pltpu
from jax.experimental.pallas import tpu_sc as plsc

_VOCAB = 1000000
_EMBED = 64
_BATCH = 16384
_NEG = 20
_COLS = _NEG + 1
_W = _EMBED // 2
_QUADS = _VOCAB // 4

_NC, _NS = 2, 16
_NW = _NC * _NS
_ROWS_PER_W = _BATCH // _NW
_R = 32
_NCHUNK = _ROWS_PER_W // _R
_CNROWS = _R * _COLS
_GLEN = 16
_GPC = _CNROWS // _GLEN
_CGP = _R // _GLEN
_KSPLIT = (7, 7, 7)


def _pack_table(t):
    tb = t.astype(jnp.bfloat16).reshape(_VOCAB, _W, 2)
    ti = lax.bitcast_convert_type(tb, jnp.int32)
    return ti.reshape(_QUADS, 128)


def _sc_scores(center_idx, cn_idx, itab, otab):
    mesh = plsc.VectorSubcoreMesh(core_axis_name="c", subcore_axis_name="s")

    @functools.partial(
        pl.kernel,
        out_type=jax.ShapeDtypeStruct((_NW, _NCHUNK, _COLS, _R), jnp.float32),
        mesh=mesh,
        scratch_types=[
            pltpu.VMEM((_CGP, _GLEN), jnp.int32),
            pltpu.VMEM((_CGP, _GLEN), jnp.int32),
            pltpu.VMEM((_GPC, _GLEN), jnp.int32),
            pltpu.VMEM((_GPC, _GLEN), jnp.int32),
            pltpu.VMEM((_R, 128), jnp.int32),
            pltpu.VMEM((_CNROWS, 128), jnp.int32),
            pltpu.VMEM((_COLS, _R), jnp.float32),
            pltpu.VMEM((_COLS, _R), jnp.float32),
            pltpu.SemaphoreType.DMA,
            pltpu.SemaphoreType.DMA,
            pltpu.SemaphoreType.DMA,
            pltpu.SemaphoreType.DMA,
            pltpu.SemaphoreType.DMA,
        ],
        compiler_params=pltpu.CompilerParams(
            use_tc_tiling_on_sc=False, needs_layout_passes=False),
    )
    def scores_kernel(cidx_hbm, cnidx_hbm, itab_hbm, otab_hbm, out_hbm,
                      cidx0, cidx1, cnidx0, cnidx1, crows, cnrows,
                      scores0, scores1, isem0, isem1, gsem, osem0, osem1):
        wid = lax.axis_index("s") * _NC + lax.axis_index("c")
        lanes = lax.iota(jnp.int32, _NS)
        cidx = (cidx0, cidx1)
        cnidx = (cnidx0, cnidx1)
        scores = (scores0, scores1)
        isem = (isem0, isem1)
        osem = (osem0, osem1)

        def idx_copies(c, p):
            return (
                pltpu.make_async_copy(cidx_hbm.at[wid, c], cidx[p], isem[p]),
                pltpu.make_async_copy(cnidx_hbm.at[wid, c], cnidx[p],
                                      isem[p]),
            )

        def fire_gathers(p):
            for j in range(_CGP):
                pltpu.make_async_copy(
                    itab_hbm.at[cidx[p][j, :] >> 2],
                    crows.at[pl.ds(j * _GLEN, _GLEN)], gsem).start()
            for j in range(_GPC):
                pltpu.make_async_copy(
                    otab_hbm.at[cnidx[p][j, :] >> 2],
                    cnrows.at[pl.ds(j * _GLEN, _GLEN)], gsem).start()

        def drain_gathers():
            pltpu.make_async_copy(
                itab_hbm.at[pl.ds(0, _R)], crows, gsem).wait()
            pltpu.make_async_copy(
                otab_hbm.at[pl.ds(0, _CNROWS)], cnrows, gsem).wait()

        def out_copy(c, p):
            return pltpu.make_async_copy(scores[p], out_hbm.at[wid, c],
                                         osem[p])

        def unpack(word):
            lo = plsc.bitcast(word << 16, jnp.float32)
            hi = plsc.bitcast(word & jnp.int32(-65536), jnp.float32)
            return lo, hi

        def compute(p):
            for g in range(_R // _NS):
                r_vec = g * _NS + lanes
                cvals = cidx[p][g, :]
                choff = (cvals & 3) << 5
                k0 = 0
                for nk in _KSPLIT:
                    ks = range(k0, k0 + nk)
                    k0 += nk
                    flat = [r_vec * _COLS + k for k in ks]
                    hoff = []
                    for fk in flat:
                        vk = plsc.load_gather(cnidx[p], [fk >> 4, fk & 15])
                        hoff.append((vk & 3) << 5)

                    def w_body(w, accs):
                        cen = plsc.load_gather(crows, [r_vec, choff + w])
                        cen_lo, cen_hi = unpack(cen)
                        out = []
                        for a, fk, hk in zip(accs, flat, hoff):
                            cw = plsc.load_gather(cnrows, [fk, hk + w])
                            lo, hi = unpack(cw)
                            out.append(a + cen_lo * lo + cen_hi * hi)
                        return tuple(out)

                    accs = lax.fori_loop(
                        0, _W, w_body,
                        tuple(jnp.zeros((_NS,), jnp.float32)
                              for _ in ks))
                    for k, acc in zip(ks, accs):
                        val = -acc if k == 0 else acc
                        scores[p][k, pl.ds(g * _NS, _NS)] = val

        for cp in idx_copies(0, 0):
            cp.start()
        for cp in idx_copies(1, 1):
            cp.start()
        out_copy(0, 0).start()
        out_copy(1, 1).start()

        def pair_body(i, carry):
            for half in range(2):
                p = half
                c = 2 * i + half
                for cp in idx_copies(c, p):
                    cp.wait()
                fire_gathers(p)
                for cp in idx_copies((c + 2) & (_NCHUNK - 1), p):
                    cp.start()
                drain_gathers()
                out_copy(c, p).wait()
                compute(p)
                out_copy(c, p).start()
            return carry

        lax.fori_loop(0, _NCHUNK // 2, pair_body, 0)

        for cp in idx_copies(0, 0):
            cp.wait()
        for cp in idx_copies(1, 1):
            cp.wait()
        out_copy(_NCHUNK - 2, 0).wait()
        out_copy(_NCHUNK - 1, 1).wait()

    return scores_kernel(center_idx, cn_idx, itab, otab)


def _loss_body(s_ref, o_ref):
    t = s_ref[...]
    sp = jnp.maximum(t, 0.0) + jnp.log1p(jnp.exp(-jnp.abs(t)))
    o_ref[0, 0] = jnp.sum(sp) * (1.0 / _BATCH)


def kernel(center, context, negative, input_table, output_table):
    cn = jnp.concatenate([context[:, None], negative], axis=1)
    cn = cn.reshape(_NW, _NCHUNK, _GPC, _GLEN).astype(jnp.int32)
    cidx = center.reshape(_NW, _NCHUNK, _CGP, _GLEN).astype(jnp.int32)

    itab = _pack_table(input_table)
    otab = _pack_table(output_table)
    scores = _sc_scores(cidx, cn, itab, otab)

    flat = scores.reshape(_BATCH * _COLS // 128, 128)
    loss = pl.pallas_call(
        _loss_body,
        out_shape=jax.ShapeDtypeStruct((1, 1), jnp.float32),
        out_specs=pl.BlockSpec(memory_space=pltpu.SMEM),
    )(flat)
    return loss[0, 0]


# speedup vs baseline: 2.2609x; 2.2609x over previous
import functools

import jax
import jax.numpy as jnp
from jax import lax
from jax.experimental import pallas as pl
from jax.experimental.pallas import tpu as pltpu
from jax.experimental.pallas import tpu_sc as plsc

_VOCAB = 1000000
_EMBED = 64
_BATCH = 16384
_NEG = 20
_COLS = _NEG + 1

_NC, _NS = 2, 16
_NW = _NC * _NS
_ROWS_PER_W = _BATCH // _NW
_R = 32
_NCHUNK = _ROWS_PER_W // _R
_CNROWS = _R * _COLS
_GLEN = 16
_GPC = _CNROWS // _GLEN
_CGP = _R // _GLEN
_KSPLIT = (7, 7, 7)


def _sc_scores(center_idx, cn_idx, input_table, output_table):
    mesh = plsc.VectorSubcoreMesh(core_axis_name="c", subcore_axis_name="s")

    @functools.partial(
        pl.kernel,
        out_type=jax.ShapeDtypeStruct((_NW, _NCHUNK, _COLS, _R), jnp.float32),
        mesh=mesh,
        scratch_types=[
            pltpu.VMEM((_CGP, _GLEN), jnp.int32),
            pltpu.VMEM((_CGP, _GLEN), jnp.int32),
            pltpu.VMEM((_GPC, _GLEN), jnp.int32),
            pltpu.VMEM((_GPC, _GLEN), jnp.int32),
            pltpu.VMEM((_R, _EMBED), jnp.float32),
            pltpu.VMEM((_CNROWS, _EMBED), jnp.float32),
            pltpu.VMEM((_COLS, _R), jnp.float32),
            pltpu.VMEM((_COLS, _R), jnp.float32),
            pltpu.SemaphoreType.DMA,
            pltpu.SemaphoreType.DMA,
            pltpu.SemaphoreType.DMA,
            pltpu.SemaphoreType.DMA,
            pltpu.SemaphoreType.DMA,
        ],
        compiler_params=pltpu.CompilerParams(
            use_tc_tiling_on_sc=False, needs_layout_passes=False),
    )
    def scores_kernel(cidx_hbm, cnidx_hbm, itab_hbm, otab_hbm, out_hbm,
                      cidx0, cidx1, cnidx0, cnidx1, crows, cnrows,
                      scores0, scores1, isem0, isem1, gsem, osem0, osem1):
        wid = lax.axis_index("s") * _NC + lax.axis_index("c")
        lanes = lax.iota(jnp.int32, _NS)
        cidx = (cidx0, cidx1)
        cnidx = (cnidx0, cnidx1)
        scores = (scores0, scores1)
        isem = (isem0, isem1)
        osem = (osem0, osem1)

        def idx_copies(c, p):
            return (
                pltpu.make_async_copy(cidx_hbm.at[wid, c], cidx[p], isem[p]),
                pltpu.make_async_copy(cnidx_hbm.at[wid, c], cnidx[p],
                                      isem[p]),
            )

        def fire_gathers(p):
            for j in range(_CGP):
                pltpu.make_async_copy(
                    itab_hbm.at[cidx[p][j, :]],
                    crows.at[pl.ds(j * _GLEN, _GLEN)], gsem).start()
            for j in range(_GPC):
                pltpu.make_async_copy(
                    otab_hbm.at[cnidx[p][j, :]],
                    cnrows.at[pl.ds(j * _GLEN, _GLEN)], gsem).start()

        def drain_gathers():
            pltpu.make_async_copy(
                itab_hbm.at[pl.ds(0, _R)], crows, gsem).wait()
            pltpu.make_async_copy(
                otab_hbm.at[pl.ds(0, _CNROWS)], cnrows, gsem).wait()

        def out_copy(c, p):
            return pltpu.make_async_copy(scores[p], out_hbm.at[wid, c],
                                         osem[p])

        def compute(p):
            for g in range(_R // _NS):
                r_vec = g * _NS + lanes
                k0 = 0
                for nk in _KSPLIT:
                    ks = range(k0, k0 + nk)
                    k0 += nk
                    cn_rows = [r_vec * _COLS + k for k in ks]

                    def d_body(d, accs):
                        d_vec = jnp.full((_NS,), d, jnp.int32)
                        cen = plsc.load_gather(crows, [r_vec, d_vec])
                        return tuple(
                            a + cen * plsc.load_gather(cnrows, [rk, d_vec])
                            for a, rk in zip(accs, cn_rows)
                        )

                    accs = lax.fori_loop(
                        0, _EMBED, d_body,
                        tuple(jnp.zeros((_NS,), jnp.float32)
                              for _ in ks))
                    for k, acc in zip(ks, accs):
                        val = -acc if k == 0 else acc
                        scores[p][k, pl.ds(g * _NS, _NS)] = val

        for cp in idx_copies(0, 0):
            cp.start()
        for cp in idx_copies(1, 1):
            cp.start()
        out_copy(0, 0).start()
        out_copy(1, 1).start()

        def pair_body(i, carry):
            for half in range(2):
                p = half
                c = 2 * i + half
                for cp in idx_copies(c, p):
                    cp.wait()
                fire_gathers(p)
                for cp in idx_copies((c + 2) & (_NCHUNK - 1), p):
                    cp.start()
                drain_gathers()
                out_copy(c, p).wait()
                compute(p)
                out_copy(c, p).start()
            return carry

        lax.fori_loop(0, _NCHUNK // 2, pair_body, 0)

        for cp in idx_copies(0, 0):
            cp.wait()
        for cp in idx_copies(1, 1):
            cp.wait()
        out_copy(_NCHUNK - 2, 0).wait()
        out_copy(_NCHUNK - 1, 1).wait()

    return scores_kernel(center_idx, cn_idx, input_table, output_table)


def _loss_body(s_ref, o_ref):
    t = s_ref[...]
    sp = jnp.maximum(t, 0.0) + jnp.log1p(jnp.exp(-jnp.abs(t)))
    o_ref[0, 0] = jnp.sum(sp) * (1.0 / _BATCH)


def kernel(center, context, negative, input_table, output_table):
    cn = jnp.concatenate([context[:, None], negative], axis=1)
    cn = cn.reshape(_NW, _NCHUNK, _GPC, _GLEN).astype(jnp.int32)
    cidx = center.reshape(_NW, _NCHUNK, _CGP, _GLEN).astype(jnp.int32)

    scores = _sc_scores(cidx, cn, input_table, output_table)

    flat = scores.reshape(_BATCH * _COLS // 128, 128)
    loss = pl.pallas_call(
        _loss_body,
        out_shape=jax.ShapeDtypeStruct((1, 1), jnp.float32),
        out_specs=pl.BlockSpec(memory_space=pltpu.SMEM),
    )(flat)
    return loss[0, 0]


# speedup vs baseline: 2.3917x; 1.0578x over previous
import functools

import jax
import jax.numpy as jnp
from jax import lax
from jax.experimental import pallas as pl
from jax.experimental.pallas import tpu as pltpu
from jax.experimental.pallas import tpu_sc as plsc

_VOCAB = 1000000
_EMBED = 64
_BATCH = 16384
_NEG = 20
_COLS = _NEG + 1

_NC, _NS = 2, 16
_NW = _NC * _NS
_ROWS_PER_W = _BATCH // _NW
_R = 32
_NCHUNK = _ROWS_PER_W // _R
_CNROWS = _R * _COLS
_GLEN = 16
_GPC = _CNROWS // _GLEN
_CGP = _R // _GLEN
_KSPLIT = (7, 7, 7)
_DUNROLL = 4


def _sc_scores(center_idx, cn_idx, input_table, output_table):
    mesh = plsc.VectorSubcoreMesh(core_axis_name="c", subcore_axis_name="s")

    @functools.partial(
        pl.kernel,
        out_type=jax.ShapeDtypeStruct((_NW, _NCHUNK, _COLS, _R), jnp.float32),
        mesh=mesh,
        scratch_types=[
            pltpu.VMEM((_CGP, _GLEN), jnp.int32),
            pltpu.VMEM((_CGP, _GLEN), jnp.int32),
            pltpu.VMEM((_GPC, _GLEN), jnp.int32),
            pltpu.VMEM((_GPC, _GLEN), jnp.int32),
            pltpu.VMEM((_R, _EMBED), jnp.float32),
            pltpu.VMEM((_R, _EMBED), jnp.float32),
            pltpu.VMEM((_CNROWS, _EMBED), jnp.float32),
            pltpu.VMEM((_CNROWS, _EMBED), jnp.float32),
            pltpu.VMEM((_COLS, _R), jnp.float32),
            pltpu.VMEM((_COLS, _R), jnp.float32),
            pltpu.SemaphoreType.DMA,
            pltpu.SemaphoreType.DMA,
            pltpu.SemaphoreType.DMA,
            pltpu.SemaphoreType.DMA,
            pltpu.SemaphoreType.DMA,
            pltpu.SemaphoreType.DMA,
        ],
        compiler_params=pltpu.CompilerParams(
            use_tc_tiling_on_sc=False, needs_layout_passes=False),
    )
    def scores_kernel(cidx_hbm, cnidx_hbm, itab_hbm, otab_hbm, out_hbm,
                      cidx0, cidx1, cnidx0, cnidx1, crows0, crows1,
                      cnrows0, cnrows1, scores0, scores1,
                      isem0, isem1, gsem0, gsem1, osem0, osem1):
        wid = lax.axis_index("s") * _NC + lax.axis_index("c")
        lanes = lax.iota(jnp.int32, _NS)
        cidx = (cidx0, cidx1)
        cnidx = (cnidx0, cnidx1)
        crows = (crows0, crows1)
        cnrows = (cnrows0, cnrows1)
        scores = (scores0, scores1)
        isem = (isem0, isem1)
        gsem = (gsem0, gsem1)
        osem = (osem0, osem1)

        def idx_copies(c, p):
            return (
                pltpu.make_async_copy(cidx_hbm.at[wid, c], cidx[p], isem[p]),
                pltpu.make_async_copy(cnidx_hbm.at[wid, c], cnidx[p],
                                      isem[p]),
            )

        def fire_gathers(p):
            for j in range(_CGP):
                pltpu.make_async_copy(
                    itab_hbm.at[cidx[p][j, :]],
                    crows[p].at[pl.ds(j * _GLEN, _GLEN)], gsem[p]).start()
            for j in range(_GPC):
                pltpu.make_async_copy(
                    otab_hbm.at[cnidx[p][j, :]],
                    cnrows[p].at[pl.ds(j * _GLEN, _GLEN)], gsem[p]).start()

        def drain_gathers(p):
            pltpu.make_async_copy(
                itab_hbm.at[pl.ds(0, _R)], crows[p], gsem[p]).wait()
            pltpu.make_async_copy(
                otab_hbm.at[pl.ds(0, _CNROWS)], cnrows[p], gsem[p]).wait()

        def out_copy(c, p):
            return pltpu.make_async_copy(scores[p], out_hbm.at[wid, c],
                                         osem[p])

        def compute(p):
            for g in range(_R // _NS):
                r_vec = g * _NS + lanes
                k0 = 0
                for nk in _KSPLIT:
                    ks = range(k0, k0 + nk)
                    k0 += nk
                    cn_rows = [r_vec * _COLS + k for k in ks]

                    def d_body(dd, accs):
                        out = list(accs)
                        for du in range(_DUNROLL):
                            d_vec = jnp.full((_NS,), dd * _DUNROLL + du,
                                             jnp.int32)
                            cen = plsc.load_gather(crows[p], [r_vec, d_vec])
                            out = [
                                a + cen * plsc.load_gather(
                                    cnrows[p], [rk, d_vec])
                                for a, rk in zip(out, cn_rows)
                            ]
                        return tuple(out)

                    accs = lax.fori_loop(
                        0, _EMBED // _DUNROLL, d_body,
                        tuple(jnp.zeros((_NS,), jnp.float32)
                              for _ in ks))
                    for k, acc in zip(ks, accs):
                        val = -acc if k == 0 else acc
                        scores[p][k, pl.ds(g * _NS, _NS)] = val

        for cp in idx_copies(0, 0):
            cp.start()
        for cp in idx_copies(0, 0):
            cp.wait()
        fire_gathers(0)
        for cp in idx_copies(1, 1):
            cp.start()
        for cp in idx_copies(2, 0):
            cp.start()
        out_copy(0, 0).start()
        out_copy(1, 1).start()

        def pair_body(i, carry):
            for half in range(2):
                p = half
                c = 2 * i + half
                cn = (c + 1) & (_NCHUNK - 1)
                for cp in idx_copies(cn, 1 - p):
                    cp.wait()
                fire_gathers(1 - p)
                for cp in idx_copies((c + 3) & (_NCHUNK - 1), 1 - p):
                    cp.start()
                drain_gathers(p)
                out_copy(c, p).wait()
                compute(p)
                out_copy(c, p).start()
            return carry

        lax.fori_loop(0, _NCHUNK // 2, pair_body, 0)

        drain_gathers(0)
        for cp in idx_copies(2, 0):
            cp.wait()
        for cp in idx_copies(1, 1):
            cp.wait()
        out_copy(_NCHUNK - 2, 0).wait()
        out_copy(_NCHUNK - 1, 1).wait()

    return scores_kernel(center_idx, cn_idx, input_table, output_table)


def _loss_body(s_ref, o_ref):
    t = s_ref[...]
    sp = jnp.maximum(t, 0.0) + jnp.log1p(jnp.exp(-jnp.abs(t)))
    o_ref[0, 0] = jnp.sum(sp) * (1.0 / _BATCH)


def kernel(center, context, negative, input_table, output_table):
    cn = jnp.concatenate([context[:, None], negative], axis=1)
    cn = cn.reshape(_NW, _NCHUNK, _GPC, _GLEN).astype(jnp.int32)
    cidx = center.reshape(_NW, _NCHUNK, _CGP, _GLEN).astype(jnp.int32)

    scores = _sc_scores(cidx, cn, input_table, output_table)

    flat = scores.reshape(_BATCH * _COLS // 128, 128)
    loss = pl.pallas_call(
        _loss_body,
        out_shape=jax.ShapeDtypeStruct((1, 1), jnp.float32),
        out_specs=pl.BlockSpec(memory_space=pltpu.SMEM),
    )(flat)
    return loss[0, 0]
